# speedup vs baseline: 1.1294x; 1.1294x over previous
import functools

import jax
import jax.numpy as jnp
from jax import lax
from jax.experimental import pallas as pl
from jax.experimental.pallas import tpu as pltpu
from jax.experimental.pallas import tpu_sc as plsc

NC = 2
NS = 16
NW = NC * NS
LANES = 16
K = 16


def _build_agg(n_pad, d, npc):
    ipc = npc * K
    npw = n_pad // NW
    nch = npw // npc

    mesh = plsc.VectorSubcoreMesh(
        core_axis_name="c", subcore_axis_name="s",
        num_cores=NC, num_subcores=NS)

    def body(x_hbm, adjs_hbm, out_hbm, idx_v, rows_v, stage_v, sem):
        wid = lax.axis_index("s") * NC + lax.axis_index("c")
        pltpu.sync_copy(adjs_hbm.at[pl.ds(wid * nch, nch)], idx_v)

        def chunk_body(j, _):
            pltpu.async_copy(x_hbm.at[idx_v.at[j]], rows_v, sem).wait()

            def slice_body(si, _):
                col = pl.ds(si * LANES, LANES)
                for n in range(npc):
                    v = rows_v[n * K, col]
                    for k in range(1, K):
                        v = v + rows_v[n * K + k, col]
                    stage_v[n, col] = v * (1.0 / K)
                return 0

            lax.fori_loop(0, d // LANES, slice_body, 0, unroll=False)
            pltpu.sync_copy(
                stage_v, out_hbm.at[pl.ds(wid * npw + j * npc, npc)])
            return 0

        lax.fori_loop(0, nch, chunk_body, 0, unroll=False)

    return pl.kernel(
        body,
        out_type=jax.ShapeDtypeStruct((n_pad, d), jnp.float32),
        mesh=mesh,
        scratch_types=[
            pltpu.VMEM((nch, ipc), jnp.int32),
            pltpu.VMEM((ipc, d), jnp.float32),
            pltpu.VMEM((npc, d), jnp.float32),
            pltpu.SemaphoreType.DMA,
        ],
    )


def _mm_kernel(x_ref, a_ref, ws_ref, wn_ref, o_ref):
    acc = jnp.dot(x_ref[...], ws_ref[...], preferred_element_type=jnp.float32)
    acc = acc + jnp.dot(a_ref[...], wn_ref[...],
                        preferred_element_type=jnp.float32)
    o_ref[...] = jnp.maximum(acc, 0.0)


@functools.partial(jax.jit, static_argnames=("bm",))
def _mm(x, agg, ws, wn, bm=512):
    m, d = x.shape
    h = ws.shape[1]
    return pl.pallas_call(
        _mm_kernel,
        grid=(m // bm,),
        in_specs=[
            pl.BlockSpec((bm, d), lambda i: (i, 0)),
            pl.BlockSpec((bm, d), lambda i: (i, 0)),
            pl.BlockSpec((d, h), lambda i: (0, 0)),
            pl.BlockSpec((d, h), lambda i: (0, 0)),
        ],
        out_specs=pl.BlockSpec((bm, h), lambda i: (i, 0)),
        out_shape=jax.ShapeDtypeStruct((m, h), jnp.float32),
    )(x, agg, ws, wn)


def kernel(nodes, adjs, features, W1, W2):
    n, k = adjs.shape
    d_in = features.shape[1]
    h1_dim = W1.shape[1]
    h2_dim = W2.shape[1]

    npc1 = 2048 // d_in
    npc2 = 2048 // h1_dim

    quantum = NW * max(npc1, npc2)
    n_pad = ((n + quantum - 1) // quantum) * quantum

    adjs_flat = jnp.pad(adjs, ((0, n_pad - n), (0, 0))).reshape(-1)
    adjs_r1 = adjs_flat.reshape(-1, npc1 * K)
    adjs_r2 = adjs_flat.reshape(-1, npc2 * K)

    agg1 = _build_agg(n_pad, d_in, npc1)(features, adjs_r1)
    x_pad = jnp.pad(features, ((0, n_pad - n), (0, 0)))
    h1 = _mm(x_pad, agg1, W1[:d_in], W1[d_in:])

    agg2 = _build_agg(n_pad, h1_dim, npc2)(h1, adjs_r2)
    h2 = _mm(h1, agg2, W2[:h1_dim], W2[h1_dim:])
    return h2[:n]


# speedup vs baseline: 1.5151x; 1.3414x over previous
import functools

import jax
import jax.numpy as jnp
from jax import lax
from jax.experimental import pallas as pl
from jax.experimental.pallas import tpu as pltpu
from jax.experimental.pallas import tpu_sc as plsc

NC = 2
NS = 16
NW = NC * NS
LANES = 16
K = 16


def _build_agg(n_pad, d, npc):
    ipc = npc * K
    npw = n_pad // NW
    nch = npw // npc

    mesh = plsc.VectorSubcoreMesh(
        core_axis_name="c", subcore_axis_name="s",
        num_cores=NC, num_subcores=NS)

    assert nch % 2 == 0

    def body(x_hbm, adjs_hbm, out_hbm, idx_v,
             rows0, rows1, stg0, stg1, gsem0, gsem1, osem0, osem1):
        wid = lax.axis_index("s") * NC + lax.axis_index("c")
        rows = (rows0, rows1)
        stg = (stg0, stg1)
        gsem = (gsem0, gsem1)
        osem = (osem0, osem1)
        pltpu.sync_copy(adjs_hbm.at[pl.ds(wid * nch, nch)], idx_v)

        def gather(j, b):
            return pltpu.make_async_copy(
                x_hbm.at[idx_v.at[j]], rows[b], gsem[b])

        def store(j, b):
            return pltpu.make_async_copy(
                stg[b], out_hbm.at[pl.ds(wid * npw + j * npc, npc)], osem[b])

        gather(0, 0).start()
        gather(1, 1).start()

        def outer(j2, _):
            for b in range(2):
                j = j2 * 2 + b
                gather(j, b).wait()

                @pl.when(j2 > 0)
                def _():
                    store(j - 2, b).wait()

                def slice_body(si, _):
                    col = pl.ds(si * LANES, LANES)
                    for n in range(npc):
                        v = rows[b][n * K, col]
                        for k in range(1, K):
                            v = v + rows[b][n * K + k, col]
                        stg[b][n, col] = v * (1.0 / K)
                    return 0

                lax.fori_loop(0, d // LANES, slice_body, 0, unroll=False)

                @pl.when(j + 2 < nch)
                def _():
                    gather(j + 2, b).start()

                store(j, b).start()
            return 0

        lax.fori_loop(0, nch // 2, outer, 0, unroll=False)
        store(nch - 2, 0).wait()
        store(nch - 1, 1).wait()

    return pl.kernel(
        body,
        out_type=jax.ShapeDtypeStruct((n_pad, d), jnp.float32),
        mesh=mesh,
        scratch_types=[
            pltpu.VMEM((nch, ipc), jnp.int32),
            pltpu.VMEM((ipc, d), jnp.float32),
            pltpu.VMEM((ipc, d), jnp.float32),
            pltpu.VMEM((npc, d), jnp.float32),
            pltpu.VMEM((npc, d), jnp.float32),
            pltpu.SemaphoreType.DMA,
            pltpu.SemaphoreType.DMA,
            pltpu.SemaphoreType.DMA,
            pltpu.SemaphoreType.DMA,
        ],
    )


def _mm_kernel(x_ref, a_ref, ws_ref, wn_ref, o_ref):
    acc = jnp.dot(x_ref[...], ws_ref[...], preferred_element_type=jnp.float32)
    acc = acc + jnp.dot(a_ref[...], wn_ref[...],
                        preferred_element_type=jnp.float32)
    o_ref[...] = jnp.maximum(acc, 0.0)


@functools.partial(jax.jit, static_argnames=("bm",))
def _mm(x, agg, ws, wn, bm=512):
    m, d = x.shape
    h = ws.shape[1]
    return pl.pallas_call(
        _mm_kernel,
        grid=(m // bm,),
        in_specs=[
            pl.BlockSpec((bm, d), lambda i: (i, 0)),
            pl.BlockSpec((bm, d), lambda i: (i, 0)),
            pl.BlockSpec((d, h), lambda i: (0, 0)),
            pl.BlockSpec((d, h), lambda i: (0, 0)),
        ],
        out_specs=pl.BlockSpec((bm, h), lambda i: (i, 0)),
        out_shape=jax.ShapeDtypeStruct((m, h), jnp.float32),
    )(x, agg, ws, wn)


def kernel(nodes, adjs, features, W1, W2):
    n, k = adjs.shape
    d_in = features.shape[1]
    h1_dim = W1.shape[1]
    h2_dim = W2.shape[1]

    npc1 = 2048 // d_in
    npc2 = 2048 // h1_dim

    quantum = NW * max(npc1, npc2)
    n_pad = ((n + quantum - 1) // quantum) * quantum

    adjs_flat = jnp.pad(adjs, ((0, n_pad - n), (0, 0))).reshape(-1)
    adjs_r1 = adjs_flat.reshape(-1, npc1 * K)
    adjs_r2 = adjs_flat.reshape(-1, npc2 * K)

    agg1 = _build_agg(n_pad, d_in, npc1)(features, adjs_r1)
    x_pad = jnp.pad(features, ((0, n_pad - n), (0, 0)))
    h1 = _mm(x_pad, agg1, W1[:d_in], W1[d_in:])

    agg2 = _build_agg(n_pad, h1_dim, npc2)(h1, adjs_r2)
    h2 = _mm(h1, agg2, W2[:h1_dim], W2[h1_dim:])
    return h2[:n]
